# baseline (speedup 1.0000x reference)
import jax
import jax.numpy as jnp
from jax import lax
from jax.experimental import pallas as pl
from jax.experimental.pallas import tpu as pltpu
from jax.experimental.pallas import tpu_sc as plsc

VOCAB = 100000
EMB = 128
B = 16384
K = 50

NC = 2
NS = 16
NW = NC * NS
NB = B // NW
KP = 56
T = 2
NCHUNK = NB // T


def _sc_body(wout_hbm, win_hbm, idx_hbm, center_hbm, score_out,
             cidx, crows, gidx, rows, score_buf, sem):
    wid = lax.axis_index("s") * NC + lax.axis_index("c")
    base = wid * NB
    lane = jnp.arange(16, dtype=jnp.int32)

    for j in range(4):
        pltpu.sync_copy(center_hbm.at[pl.ds(base + j * 128, 128)], cidx.at[j])
    for j in range(4):
        pltpu.async_copy(win_hbm.at[cidx.at[j]],
                         crows.at[pl.ds(j * 128, 128)], sem).wait()

    def chunk(g, _):
        off = base * KP + g * (T * KP)
        pltpu.sync_copy(idx_hbm.at[pl.ds(off, T * KP)], gidx)
        pltpu.async_copy(wout_hbm.at[gidx], rows, sem).wait()
        for i in range(T):
            ii = g * T + i
            c = [crows[ii, pl.ds(16 * j, 16)] for j in range(8)]
            for grp in range(4):
                vec = jnp.zeros((16,), jnp.float32)
                for t in range(16 if grp < 3 else 8):
                    r = i * KP + grp * 16 + t
                    acc = rows[r, pl.ds(0, 16)] * c[0]
                    for j in range(1, 8):
                        acc = acc + rows[r, pl.ds(16 * j, 16)] * c[j]
                    vec = jnp.where(lane == t, jnp.sum(acc), vec)
                score_buf[pl.ds(ii * 64 + grp * 16, 16)] = vec
        return 0

    lax.fori_loop(0, NCHUNK, chunk, 0)

    pltpu.sync_copy(score_buf, score_out.at[pl.ds(base * 64, NB * 64)])


def _loss_body(s_ref, w_ref, out_ref):
    s = s_ref[...]
    w = w_ref[...]
    lane = lax.broadcasted_iota(jnp.int32, s.shape, 1)
    sig = lambda x: 1.0 / (1.0 + jnp.exp(-x))
    pos_l = jnp.where(lane == 0, jnp.log(sig(s) + 1e-10) * w, 0.0)
    neg_l = jnp.where((lane >= 1) & (lane <= K), jnp.log(sig(-s) + 1e-10), 0.0)
    out_ref[...] = jnp.reshape(-(jnp.sum(pos_l) + jnp.sum(neg_l)) / B, (1, 1))


@jax.jit
def kernel(center, context, negatives, weights, W_in, W_out):
    center = center.astype(jnp.int32)
    idx = jnp.concatenate(
        [context.astype(jnp.int32)[:, None], negatives.astype(jnp.int32)],
        axis=1)
    idx = jnp.pad(idx, ((0, 0), (0, KP - (K + 1)))).reshape(-1)

    mesh = plsc.VectorSubcoreMesh(core_axis_name="c", subcore_axis_name="s",
                                  num_cores=NC, num_subcores=NS)
    scores = pl.kernel(
        _sc_body,
        out_type=jax.ShapeDtypeStruct((B * 64,), jnp.float32),
        mesh=mesh,
        scratch_types=[
            pltpu.VMEM((4, 128), jnp.int32),
            pltpu.VMEM((NB, EMB), jnp.float32),
            pltpu.VMEM((T * KP,), jnp.int32),
            pltpu.VMEM((T * KP, EMB), jnp.float32),
            pltpu.VMEM((NB * 64,), jnp.float32),
            pltpu.SemaphoreType.DMA,
        ],
        compiler_params=pltpu.CompilerParams(needs_layout_passes=False),
    )(W_out, W_in, idx, center)

    loss = pl.pallas_call(
        _loss_body,
        out_shape=jax.ShapeDtypeStruct((1, 1), jnp.float32),
    )(scores.reshape(B, 64), weights.reshape(B, 1))
    return loss[0, 0]
